# Initial kernel scaffold; baseline (speedup 1.0000x reference)
import jax
import jax.numpy as jnp
from jax.experimental import pallas as pl


def kernel(input_, weight):
    raise NotImplementedError("write your pallas kernel here")


# speedup vs baseline: 1.8440x; 1.8440x over previous
import functools

import jax
import jax.numpy as jnp
from jax import lax
from jax.experimental import pallas as pl
from jax.experimental.pallas import tpu as pltpu
from jax.experimental.pallas import tpu_sc as plsc

NUM_EMBEDDINGS = 1000000
EMBEDDING_DIM = 64
BATCH = 16384
HIST = 50

NC, NS = 2, 16
NW = NC * NS
B = BATCH * HIST
D = EMBEDDING_DIM
RPW = B // NW
TI = 128
KS = 8
CHUNK = TI * KS
NCHUNK = RPW // CHUNK

_mesh = plsc.VectorSubcoreMesh(core_axis_name="c", subcore_axis_name="s",
                               num_cores=NC, num_subcores=NS)


@functools.partial(
    pl.kernel,
    out_type=jax.ShapeDtypeStruct((B, D), jnp.float32),
    mesh=_mesh,
    compiler_params=pltpu.CompilerParams(use_tc_tiling_on_sc=False),
    scratch_types=[
        pltpu.VMEM((KS, TI), jnp.int32),
        pltpu.VMEM((CHUNK, D), jnp.float32),
        pltpu.SemaphoreType.DMA,
    ],
)
def _embed_sc(idx_hbm, table_hbm, out_hbm, idx_v, rows_v, gsem):
    wid = lax.axis_index("s") * NC + lax.axis_index("c")
    row0 = wid * RPW
    t0 = row0 // TI

    def chunk_body(g, carry):
        off = pl.multiple_of(row0 + g * CHUNK, CHUNK)
        pltpu.sync_copy(idx_hbm.at[pl.ds(pl.multiple_of(t0 + g * KS, KS), KS)],
                        idx_v)
        copies = []
        for j in range(KS):
            copies.append(
                pltpu.async_copy(table_hbm.at[idx_v.at[j]],
                                 rows_v.at[pl.ds(j * TI, TI)], gsem))
        for c in copies:
            c.wait()
        pltpu.sync_copy(rows_v, out_hbm.at[pl.ds(off, CHUNK)])
        return carry

    lax.fori_loop(0, NCHUNK, chunk_body, 0)


def kernel(input_, weight):
    idx2d = input_.reshape(B // TI, TI)
    out = _embed_sc(idx2d, weight)
    return out.reshape(BATCH, HIST, D)


# speedup vs baseline: 1.8699x; 1.0140x over previous
import functools

import jax
import jax.numpy as jnp
from jax import lax
from jax.experimental import pallas as pl
from jax.experimental.pallas import tpu as pltpu
from jax.experimental.pallas import tpu_sc as plsc

NUM_EMBEDDINGS = 1000000
EMBEDDING_DIM = 64
BATCH = 16384
HIST = 50

NC, NS = 2, 16
NW = NC * NS
B = BATCH * HIST
D = EMBEDDING_DIM
RPW = B // NW
TI = 128
KS = 4
HC = TI * KS
NH = RPW // HC
NB = NH // 2
IDX_ROWS = RPW // TI

_mesh = plsc.VectorSubcoreMesh(core_axis_name="c", subcore_axis_name="s",
                               num_cores=NC, num_subcores=NS)


@functools.partial(
    pl.kernel,
    out_type=jax.ShapeDtypeStruct((B, D), jnp.float32),
    mesh=_mesh,
    compiler_params=pltpu.CompilerParams(use_tc_tiling_on_sc=False),
    scratch_types=[
        pltpu.VMEM((IDX_ROWS, TI), jnp.int32),
        pltpu.VMEM((2, HC, D), jnp.float32),
        pltpu.SemaphoreType.DMA,
        pltpu.SemaphoreType.DMA,
        pltpu.SemaphoreType.DMA,
        pltpu.SemaphoreType.DMA,
    ],
)
def _embed_sc(idx_hbm, table_hbm, out_hbm, idx_v, rows_v, g0, g1, o0, o1):
    wid = lax.axis_index("s") * NC + lax.axis_index("c")
    row0 = wid * RPW
    t0 = wid * IDX_ROWS

    pltpu.sync_copy(idx_hbm.at[pl.ds(pl.multiple_of(t0, 8), IDX_ROWS)], idx_v)

    gsems = (g0, g1)
    osems = (o0, o1)

    def fire_gathers(h, slot):
        for j in range(KS):
            pltpu.async_copy(table_hbm.at[idx_v.at[KS * h + j]],
                             rows_v.at[slot, pl.ds(j * TI, TI)], gsems[slot])

    def drain_gathers(slot):
        pltpu.make_async_copy(table_hbm.at[pl.ds(0, HC)],
                              rows_v.at[slot], gsems[slot]).wait()

    def fire_out(h, slot):
        off = pl.multiple_of(row0 + h * HC, HC)
        pltpu.async_copy(rows_v.at[slot], out_hbm.at[pl.ds(off, HC)],
                         osems[slot])

    def drain_out(slot):
        pltpu.make_async_copy(rows_v.at[slot],
                              out_hbm.at[pl.ds(0, HC)], osems[slot]).wait()

    fire_gathers(0, 0)

    def body(g, carry):
        h0 = 2 * g
        h1 = 2 * g + 1
        drain_gathers(0)
        fire_out(h0, 0)

        @pl.when(g >= 1)
        def _():
            drain_out(1)
        fire_gathers(h1, 1)

        drain_gathers(1)
        fire_out(h1, 1)

        @pl.when(g + 1 < NB)
        def _():
            drain_out(0)
            fire_gathers(h1 + 1, 0)
        return carry

    lax.fori_loop(0, NB, body, 0)
    drain_out(0)
    drain_out(1)


def kernel(input_, weight):
    idx2d = input_.reshape(B // TI, TI)
    out = _embed_sc(idx2d, weight)
    return out.reshape(BATCH, HIST, D)
